# Initial kernel scaffold; baseline (speedup 1.0000x reference)
import jax
import jax.numpy as jnp
from jax.experimental import pallas as pl


def kernel(emotion_ids, table, W, b):
    raise NotImplementedError("write your pallas kernel here")


# speedup vs baseline: 5.2616x; 5.2616x over previous
import functools

import jax
import jax.numpy as jnp
from jax import lax
from jax.experimental import pallas as pl
from jax.experimental.pallas import tpu as pltpu
from jax.experimental.pallas import tpu_sc as plsc


def _proj_body(tab_ref, w_ref, b_ref, out_ref):
    out_ref[...] = lax.dot_general(
        tab_ref[...], w_ref[...],
        dimension_numbers=(((1,), (1,)), ((), ())),
        preferred_element_type=jnp.float32,
    ) + b_ref[...]


def _project_table(table, W, b):
    V, D = table.shape
    R = 2000
    assert V % R == 0
    return pl.pallas_call(
        _proj_body,
        grid=(V // R,),
        in_specs=[
            pl.BlockSpec((R, D), lambda i: (i, 0)),
            pl.BlockSpec((D, D), lambda i: (0, 0)),
            pl.BlockSpec((1, D), lambda i: (0, 0)),
        ],
        out_specs=pl.BlockSpec((R, D), lambda i: (i, 0)),
        out_shape=jax.ShapeDtypeStruct((V, D), jnp.float32),
    )(table, W, b.reshape(1, D))


_SUB = 128
_CH = 512
_NSUB = _CH // _SUB


def _make_gather(V, D, B, nc, ns):
    nw = nc * ns
    assert B % (nw * _CH) == 0
    b_per_w = B // nw
    nch = b_per_w // _CH

    def body(tab_hbm, idx_hbm, out_hbm, idx_v, rows_v, gsem):
        wid = lax.axis_index("s") * nc + lax.axis_index("c")

        def chunk(g, carry):
            base = wid * b_per_w + g * _CH
            pltpu.sync_copy(idx_hbm.at[pl.ds(base, _CH)], idx_v)
            descs = [
                pltpu.async_copy(
                    tab_hbm.at[idx_v.at[pl.ds(j * _SUB, _SUB)]],
                    rows_v.at[pl.ds(j * _SUB, _SUB)],
                    gsem,
                )
                for j in range(_NSUB)
            ]
            for d in descs:
                d.wait()
            pltpu.sync_copy(rows_v, out_hbm.at[pl.ds(base, _CH)])
            return carry

        lax.fori_loop(0, nch, chunk, 0)

    return pl.kernel(
        body,
        out_type=jax.ShapeDtypeStruct((B, D), jnp.float32),
        mesh=plsc.VectorSubcoreMesh(core_axis_name="c", subcore_axis_name="s"),
        scratch_types=[
            pltpu.VMEM((_CH,), jnp.int32),
            pltpu.VMEM((_CH, D), jnp.float32),
            pltpu.SemaphoreType.DMA,
        ],
        compiler_params=pltpu.CompilerParams(use_tc_tiling_on_sc=False),
    )


def kernel(emotion_ids, table, W, b):
    BATCH, HIST = emotion_ids.shape
    V, D = table.shape
    B = BATCH * HIST

    table2 = _project_table(table, W, b)

    info = plsc.get_sparse_core_info()
    gather = _make_gather(V, D, B, info.num_cores, info.num_subcores)
    flat_ids = emotion_ids.reshape(B).astype(jnp.int32)
    out = gather(table2, flat_ids)
    return out.reshape(BATCH, HIST, D)


# speedup vs baseline: 5.5167x; 1.0485x over previous
import functools

import jax
import jax.numpy as jnp
from jax import lax
from jax.experimental import pallas as pl
from jax.experimental.pallas import tpu as pltpu
from jax.experimental.pallas import tpu_sc as plsc


def _proj_body(tab_ref, w_ref, b_ref, out_ref):
    out_ref[...] = lax.dot_general(
        tab_ref[...], w_ref[...],
        dimension_numbers=(((1,), (1,)), ((), ())),
        preferred_element_type=jnp.float32,
    ) + b_ref[...]


def _project_table(table, W, b):
    V, D = table.shape
    R = 2000
    assert V % R == 0
    return pl.pallas_call(
        _proj_body,
        grid=(V // R,),
        in_specs=[
            pl.BlockSpec((R, D), lambda i: (i, 0)),
            pl.BlockSpec((D, D), lambda i: (0, 0)),
            pl.BlockSpec((1, D), lambda i: (0, 0)),
        ],
        out_specs=pl.BlockSpec((R, D), lambda i: (i, 0)),
        out_shape=jax.ShapeDtypeStruct((V, D), jnp.float32),
    )(table, W, b.reshape(1, D))


_SUB = 128
_CH = 512
_NSUB = _CH // _SUB


def _make_gather(V, D, B, nc, ns):
    nw = nc * ns
    assert B % (nw * _CH) == 0
    b_per_w = B // nw
    nch = b_per_w // _CH
    assert nch % 2 == 0

    def body(tab_hbm, idx_hbm, out_hbm, idx_v, rows_v, gsem0, gsem1, ssem0, ssem1):
        wid = lax.axis_index("s") * nc + lax.axis_index("c")
        wbase = wid * b_per_w
        gsems = (gsem0, gsem1)
        ssems = (ssem0, ssem1)

        def fire_gathers(buf, gsem):
            for j in range(_NSUB):
                pltpu.async_copy(
                    tab_hbm.at[idx_v.at[buf, pl.ds(j * _SUB, _SUB)]],
                    rows_v.at[buf, pl.ds(j * _SUB, _SUB)],
                    gsem,
                )

        def drain_gather(buf):
            pltpu.make_async_copy(
                out_hbm.at[pl.ds(0, _CH)], rows_v.at[buf], gsems[buf]
            ).wait()

        def drain_scatter(buf):
            pltpu.make_async_copy(
                rows_v.at[buf], out_hbm.at[pl.ds(0, _CH)], ssems[buf]
            ).wait()

        pltpu.sync_copy(idx_hbm.at[pl.ds(wbase, _CH)], idx_v.at[0])
        fire_gathers(0, gsems[0])

        def pair(gg, carry):
            for par in (0, 1):
                g = 2 * gg + par
                nbuf = 1 - par

                @pl.when(g + 1 < nch)
                def _():
                    @pl.when(g >= 1)
                    def _():
                        drain_scatter(nbuf)

                    pltpu.sync_copy(
                        idx_hbm.at[pl.ds(wbase + (g + 1) * _CH, _CH)],
                        idx_v.at[nbuf],
                    )
                    fire_gathers(nbuf, gsems[nbuf])

                drain_gather(par)
                pltpu.async_copy(
                    rows_v.at[par],
                    out_hbm.at[pl.ds(wbase + g * _CH, _CH)],
                    ssems[par],
                )
            return carry

        lax.fori_loop(0, nch // 2, pair, 0)
        drain_scatter(0)
        drain_scatter(1)

    return pl.kernel(
        body,
        out_type=jax.ShapeDtypeStruct((B, D), jnp.float32),
        mesh=plsc.VectorSubcoreMesh(core_axis_name="c", subcore_axis_name="s"),
        scratch_types=[
            pltpu.VMEM((2, _CH), jnp.int32),
            pltpu.VMEM((2, _CH, D), jnp.float32),
            pltpu.SemaphoreType.DMA,
            pltpu.SemaphoreType.DMA,
            pltpu.SemaphoreType.DMA,
            pltpu.SemaphoreType.DMA,
        ],
        compiler_params=pltpu.CompilerParams(use_tc_tiling_on_sc=False),
    )


def kernel(emotion_ids, table, W, b):
    BATCH, HIST = emotion_ids.shape
    V, D = table.shape
    B = BATCH * HIST

    table2 = _project_table(table, W, b)

    info = plsc.get_sparse_core_info()
    gather = _make_gather(V, D, B, info.num_cores, info.num_subcores)
    flat_ids = emotion_ids.reshape(B).astype(jnp.int32)
    out = gather(table2, flat_ids)
    return out.reshape(BATCH, HIST, D)


# speedup vs baseline: 5.6209x; 1.0189x over previous
import functools

import jax
import jax.numpy as jnp
from jax import lax
from jax.experimental import pallas as pl
from jax.experimental.pallas import tpu as pltpu
from jax.experimental.pallas import tpu_sc as plsc


def _proj_body(tab_ref, w_ref, b_ref, out_ref):
    out_ref[...] = lax.dot_general(
        tab_ref[...], w_ref[...],
        dimension_numbers=(((1,), (1,)), ((), ())),
        preferred_element_type=jnp.float32,
    ) + b_ref[...]


def _project_table(table, W, b):
    V, D = table.shape
    R = 4000
    assert V % R == 0
    return pl.pallas_call(
        _proj_body,
        grid=(V // R,),
        in_specs=[
            pl.BlockSpec((R, D), lambda i: (i, 0)),
            pl.BlockSpec((D, D), lambda i: (0, 0)),
            pl.BlockSpec((1, D), lambda i: (0, 0)),
        ],
        out_specs=pl.BlockSpec((R, D), lambda i: (i, 0)),
        out_shape=jax.ShapeDtypeStruct((V, D), jnp.float32),
    )(table, W, b.reshape(1, D))


_NB = 16


def _make_gather(V, D, BATCH, HIST, nc, ns):
    nw = nc * ns
    assert BATCH % (nw * _NB) == 0
    b_per_w = BATCH // nw
    nch = b_per_w // _NB
    assert nch % 2 == 0

    def body(tab_hbm, ids_hbm, out_hbm, idx_v, rows_v, gsem0, gsem1, ssem0, ssem1):
        wid = lax.axis_index("s") * nc + lax.axis_index("c")
        wbase = wid * b_per_w
        gsems = (gsem0, gsem1)
        ssems = (ssem0, ssem1)

        def fire_gathers(buf, gsem):
            for j in range(_NB):
                pltpu.async_copy(
                    tab_hbm.at[idx_v.at[buf, j]],
                    rows_v.at[buf, j],
                    gsem,
                )

        def drain_gather(buf):
            pltpu.make_async_copy(
                out_hbm.at[pl.ds(0, _NB)], rows_v.at[buf], gsems[buf]
            ).wait()

        def drain_scatter(buf):
            pltpu.make_async_copy(
                rows_v.at[buf], out_hbm.at[pl.ds(0, _NB)], ssems[buf]
            ).wait()

        pltpu.sync_copy(ids_hbm.at[pl.ds(wbase, _NB)], idx_v.at[0])
        fire_gathers(0, gsems[0])

        def pair(gg, carry):
            for par in (0, 1):
                g = 2 * gg + par
                nbuf = 1 - par

                @pl.when(g + 1 < nch)
                def _():
                    @pl.when(g >= 1)
                    def _():
                        drain_scatter(nbuf)

                    pltpu.sync_copy(
                        ids_hbm.at[pl.ds(wbase + (g + 1) * _NB, _NB)],
                        idx_v.at[nbuf],
                    )
                    fire_gathers(nbuf, gsems[nbuf])

                drain_gather(par)
                pltpu.async_copy(
                    rows_v.at[par],
                    out_hbm.at[pl.ds(wbase + g * _NB, _NB)],
                    ssems[par],
                )
            return carry

        lax.fori_loop(0, nch // 2, pair, 0)
        drain_scatter(0)
        drain_scatter(1)

    return pl.kernel(
        body,
        out_type=jax.ShapeDtypeStruct((BATCH, HIST, D), jnp.float32),
        mesh=plsc.VectorSubcoreMesh(core_axis_name="c", subcore_axis_name="s"),
        scratch_types=[
            pltpu.VMEM((2, _NB, HIST), jnp.int32),
            pltpu.VMEM((2, _NB, HIST, D), jnp.float32),
            pltpu.SemaphoreType.DMA,
            pltpu.SemaphoreType.DMA,
            pltpu.SemaphoreType.DMA,
            pltpu.SemaphoreType.DMA,
        ],
        compiler_params=pltpu.CompilerParams(use_tc_tiling_on_sc=False),
    )


def kernel(emotion_ids, table, W, b):
    BATCH, HIST = emotion_ids.shape
    V, D = table.shape

    table2 = _project_table(table, W, b)

    info = plsc.get_sparse_core_info()
    gather = _make_gather(V, D, BATCH, HIST, info.num_cores, info.num_subcores)
    return gather(table2, emotion_ids.astype(jnp.int32))


# speedup vs baseline: 6.2002x; 1.1031x over previous
import functools

import jax
import jax.numpy as jnp
from jax import lax
from jax.experimental import pallas as pl
from jax.experimental.pallas import tpu as pltpu
from jax.experimental.pallas import tpu_sc as plsc


def _proj_body(tab_ref, w_ref, b_ref, out_ref):
    out_ref[...] = lax.dot_general(
        tab_ref[...], w_ref[...],
        dimension_numbers=(((1,), (1,)), ((), ())),
        preferred_element_type=jnp.float32,
    ) + b_ref[...]


def _project_table(table, W, b):
    V, D = table.shape
    R = 4000
    assert V % R == 0
    return pl.pallas_call(
        _proj_body,
        grid=(V // R,),
        in_specs=[
            pl.BlockSpec((R, D), lambda i: (i, 0)),
            pl.BlockSpec((D, D), lambda i: (0, 0)),
            pl.BlockSpec((1, D), lambda i: (0, 0)),
        ],
        out_specs=pl.BlockSpec((R, D), lambda i: (i, 0)),
        out_shape=jax.ShapeDtypeStruct((V, D), jnp.float32),
    )(table, W, b.reshape(1, D))


_NB = 16


def _make_gather(V, D, BATCH, HIST, nc, ns):
    nw = nc * ns
    assert BATCH % (nw * _NB) == 0
    b_per_w = BATCH // nw
    nch = b_per_w // _NB
    assert nch % 2 == 0

    def body(tab_hbm, ids_hbm, out_hbm, idx_v, rows_v, gsem0, gsem1, ssem0, ssem1):
        wid = lax.axis_index("s") * nc + lax.axis_index("c")
        wbase = wid * b_per_w
        gsems = (gsem0, gsem1)
        ssems = (ssem0, ssem1)

        def fire_gathers(buf, gsem):
            for l in range(HIST):
                pltpu.async_copy(
                    tab_hbm.at[idx_v.at[buf, l]],
                    rows_v.at[buf, l],
                    gsem,
                )

        def drain_gather(buf):
            pltpu.make_async_copy(
                out_hbm.at[:, pl.ds(0, _NB), :], rows_v.at[buf], gsems[buf]
            ).wait()

        def drain_scatter(buf):
            pltpu.make_async_copy(
                rows_v.at[buf], out_hbm.at[:, pl.ds(0, _NB), :], ssems[buf]
            ).wait()

        pltpu.sync_copy(ids_hbm.at[:, pl.ds(wbase, _NB)], idx_v.at[0])
        fire_gathers(0, gsems[0])

        def pair(gg, carry):
            for par in (0, 1):
                g = 2 * gg + par
                nbuf = 1 - par

                @pl.when(g + 1 < nch)
                def _():
                    @pl.when(g >= 1)
                    def _():
                        drain_scatter(nbuf)

                    pltpu.sync_copy(
                        ids_hbm.at[:, pl.ds(wbase + (g + 1) * _NB, _NB)],
                        idx_v.at[nbuf],
                    )
                    fire_gathers(nbuf, gsems[nbuf])

                drain_gather(par)
                pltpu.async_copy(
                    rows_v.at[par],
                    out_hbm.at[:, pl.ds(wbase + g * _NB, _NB), :],
                    ssems[par],
                )
            return carry

        lax.fori_loop(0, nch // 2, pair, 0)
        drain_scatter(0)
        drain_scatter(1)

    return pl.kernel(
        body,
        out_type=jax.ShapeDtypeStruct((HIST, BATCH, D), jnp.float32),
        mesh=plsc.VectorSubcoreMesh(core_axis_name="c", subcore_axis_name="s"),
        scratch_types=[
            pltpu.VMEM((2, HIST, _NB), jnp.int32),
            pltpu.VMEM((2, HIST, _NB, D), jnp.float32),
            pltpu.SemaphoreType.DMA,
            pltpu.SemaphoreType.DMA,
            pltpu.SemaphoreType.DMA,
            pltpu.SemaphoreType.DMA,
        ],
        compiler_params=pltpu.CompilerParams(use_tc_tiling_on_sc=False),
    )


def kernel(emotion_ids, table, W, b):
    BATCH, HIST = emotion_ids.shape
    V, D = table.shape

    table2 = _project_table(table, W, b)

    info = plsc.get_sparse_core_info()
    gather = _make_gather(V, D, BATCH, HIST, info.num_cores, info.num_subcores)
    ids_t = emotion_ids.astype(jnp.int32).T
    ot = gather(table2, ids_t)
    return jnp.transpose(ot, (1, 0, 2))
